# Initial kernel scaffold; baseline (speedup 1.0000x reference)
import jax
import jax.numpy as jnp
from jax.experimental import pallas as pl


def kernel(inputs):
    raise NotImplementedError("write your pallas kernel here")


# speedup vs baseline: 60.3153x; 60.3153x over previous
import dataclasses
import functools

import jax
import jax.numpy as jnp
from jax import lax
from jax.experimental import pallas as pl
from jax.experimental.pallas import tpu as pltpu
from jax.experimental.pallas import tpu_sc as plsc

T = 10
P = 1048576
S = 2
NB = 256
NBINS = NB * NB

NC = 2
NS = 16
CHUNK = 8192
H = P // NC
W = T * H
RANGE = W // NS
NSLOTS = NC * NS * 2

assert H % CHUNK == 0 and RANGE % CHUNK == 0


def _slot_schedule():
    by_t = {t: [] for t in range(T)}
    for c in range(NC):
        for s in range(NS):
            lo = s * RANGE
            t0 = lo // H
            r0 = lo - t0 * H
            len0 = min(RANGE, H - r0)
            wid = c * NS + s
            by_t[t0].append(wid * 2)
            if RANGE - len0 > 0:
                by_t[t0 + 1].append(wid * 2 + 1)
    return by_t


_BY_T = _slot_schedule()


def _i32(x):
    return jnp.asarray(x, jnp.int32)


def _sc_body(x_hbm, out_hbm, hist, buf):
    c = _i32(lax.axis_index("c"))
    s = _i32(lax.axis_index("s"))
    lo = s * _i32(RANGE)
    t0 = lo // _i32(H)
    r0 = lo - t0 * _i32(H)
    len0 = jnp.minimum(_i32(RANGE), _i32(H) - r0)
    n0 = len0 // _i32(CHUNK)
    n1 = (_i32(RANGE) - len0) // _i32(CHUNK)
    base_col = c * _i32(H)
    wid = c * _i32(NS) + s

    rows0 = lax.iota(jnp.int32, 16)
    col0 = jnp.zeros((16,), jnp.int32)
    col1 = jnp.ones((16,), jnp.int32)
    ones_f = jnp.ones((16,), jnp.float32)
    zeros_f = jnp.zeros((16,), jnp.float32)

    def zero_hist():
        def zbody(i, carry):
            hist[pl.ds(i * _i32(16), 16)] = zeros_f
            return carry

        lax.fori_loop(_i32(0), _i32(NBINS // 16), zbody, _i32(0))

    def process_chunk(t, start):
        pltpu.sync_copy(x_hbm.at[t, pl.ds(start, CHUNK)], buf)

        def ibody(i, carry):
            rows = rows0 + i * _i32(16)
            va = plsc.load_gather(buf, [rows, col0])
            vb = plsc.load_gather(buf, [rows, col1])
            idx = va * _i32(NB) + vb
            plsc.addupdate_scatter(hist, [idx], ones_f)
            return carry

        lax.fori_loop(_i32(0), _i32(CHUNK // 16), ibody, _i32(0))

    def run_segment(t, seg_start, nchunks, slot):
        zero_hist()

        def chunk_body(k, carry):
            process_chunk(t, seg_start + k * _i32(CHUNK))
            return carry

        lax.fori_loop(_i32(0), nchunks, chunk_body, _i32(0))
        pltpu.sync_copy(hist, out_hbm.at[slot])

    run_segment(t0, base_col + r0, n0, wid * _i32(2))

    @pl.when(n1 > 0)
    def _():
        run_segment(t0 + _i32(1), base_col, n1, wid * _i32(2) + _i32(1))


def _sc_histogram(x32):
    mesh = plsc.VectorSubcoreMesh(
        core_axis_name="c", subcore_axis_name="s",
        num_cores=NC, num_subcores=NS)
    cp = pltpu.CompilerParams(
        needs_layout_passes=False, use_tc_tiling_on_sc=False)
    f = pl.kernel(
        _sc_body,
        out_type=jax.ShapeDtypeStruct((NSLOTS, NBINS), jnp.float32),
        mesh=mesh,
        scratch_types=[
            pltpu.VMEM((NBINS,), jnp.float32),
            pltpu.VMEM((CHUNK, S), jnp.int32),
        ],
        compiler_params=cp,
    )
    return f(x32)


def _tc_entropy_body(p_ref, mi_ref, hm_ref, hj_ref):
    eps = 1e-10
    mi_a = jnp.float32(0.0)
    hm_a = jnp.float32(0.0)
    hj_a = jnp.float32(0.0)
    for t in range(T):
        sls = _BY_T[t]
        j = p_ref[sls[0]]
        for sl in sls[1:]:
            j = j + p_ref[sl]
        total = jnp.sum(j)
        m0 = jnp.sum(j, axis=1)
        m1 = jnp.sum(j, axis=0)
        pj = j / total
        hj = -jnp.sum(pj * jnp.log2(pj + eps))
        p0 = m0 / total
        p1 = m1 / total
        hm = (-jnp.sum(p0 * jnp.log2(p0 + eps))
              - jnp.sum(p1 * jnp.log2(p1 + eps)))
        mi_a += (hm - hj) / hm
        hm_a += hm
        hj_a += hj
    mi_ref[0] = mi_a / T
    hm_ref[0] = hm_a / T
    hj_ref[0] = hj_a / T


def _entropy(parts):
    return pl.pallas_call(
        _tc_entropy_body,
        out_shape=[jax.ShapeDtypeStruct((1,), jnp.float32)] * 3,
        out_specs=[pl.BlockSpec(memory_space=pltpu.SMEM)] * 3,
    )(parts)


def kernel(inputs):
    x32 = inputs.astype(jnp.int32)
    parts = _sc_histogram(x32)
    mi, hm, hj = _entropy(parts.reshape(NSLOTS, NB, NB))
    return (mi[0], hm[0], hj[0])


# speedup vs baseline: 3252.9290x; 53.9321x over previous
import jax
import jax.numpy as jnp
from jax import lax
from jax.experimental import pallas as pl
from jax.experimental.pallas import tpu as pltpu
from jax.experimental.pallas import tpu_sc as plsc

T = 10
P = 1048576
S = 2
NB = 256
NBINS = NB * NB

NC = 2
NS = 16
CHUNK = 8192
H = P // NC
W = T * H
RANGE = W // NS
NSLOTS = NC * NS * 2

assert H % CHUNK == 0 and RANGE % CHUNK == 0


def _slot_schedule():
    by_t = {t: [] for t in range(T)}
    for c in range(NC):
        for s in range(NS):
            lo = s * RANGE
            t0 = lo // H
            r0 = lo - t0 * H
            len0 = min(RANGE, H - r0)
            wid = c * NS + s
            by_t[t0].append(wid * 2)
            if RANGE - len0 > 0:
                by_t[t0 + 1].append(wid * 2 + 1)
    return by_t


_BY_T = _slot_schedule()


def _i32(x):
    return jnp.asarray(x, jnp.int32)


def _sc_body(x_hbm, out_hbm, hist, abuf, bbuf):
    c = _i32(lax.axis_index("c"))
    s = _i32(lax.axis_index("s"))
    lo = s * _i32(RANGE)
    t0 = lo // _i32(H)
    r0 = lo - t0 * _i32(H)
    len0 = jnp.minimum(_i32(RANGE), _i32(H) - r0)
    n0 = len0 // _i32(CHUNK)
    n1 = (_i32(RANGE) - len0) // _i32(CHUNK)
    base_col = c * _i32(H)
    wid = c * _i32(NS) + s

    ones_f = jnp.ones((16,), jnp.float32)
    zeros_f = jnp.zeros((16,), jnp.float32)

    def zero_hist():
        def zbody(i, carry):
            hist[pl.ds(i * _i32(16), 16)] = zeros_f
            return carry

        lax.fori_loop(_i32(0), _i32(NBINS // 16), zbody, _i32(0))

    def process_chunk(t, start):
        pltpu.sync_copy(x_hbm.at[t, _i32(0), pl.ds(start, CHUNK)], abuf)
        pltpu.sync_copy(x_hbm.at[t, _i32(1), pl.ds(start, CHUNK)], bbuf)

        def ibody(i, carry):
            off = i * _i32(16)
            va = plsc.bitcast(abuf[pl.ds(off, 16)], jnp.int32)
            vb = plsc.bitcast(bbuf[pl.ds(off, 16)], jnp.int32)
            idx = va * _i32(NB) + vb
            plsc.addupdate_scatter(hist, [idx], ones_f)
            return carry

        lax.fori_loop(_i32(0), _i32(CHUNK // 16), ibody, _i32(0))

    def run_segment(t, seg_start, nchunks, slot):
        zero_hist()

        def chunk_body(k, carry):
            process_chunk(t, seg_start + k * _i32(CHUNK))
            return carry

        lax.fori_loop(_i32(0), nchunks, chunk_body, _i32(0))
        pltpu.sync_copy(hist, out_hbm.at[slot])

    run_segment(t0, base_col + r0, n0, wid * _i32(2))

    @pl.when(n1 > 0)
    def _():
        run_segment(t0 + _i32(1), base_col, n1, wid * _i32(2) + _i32(1))


def _sc_histogram(xw):
    mesh = plsc.VectorSubcoreMesh(
        core_axis_name="c", subcore_axis_name="s",
        num_cores=NC, num_subcores=NS)
    cp = pltpu.CompilerParams(
        needs_layout_passes=False, use_tc_tiling_on_sc=False)
    f = pl.kernel(
        _sc_body,
        out_type=jax.ShapeDtypeStruct((NSLOTS, NBINS), jnp.float32),
        mesh=mesh,
        scratch_types=[
            pltpu.VMEM((NBINS,), jnp.float32),
            pltpu.VMEM((CHUNK,), jnp.uint32),
            pltpu.VMEM((CHUNK,), jnp.uint32),
        ],
        compiler_params=cp,
    )
    return f(xw)


def _tc_entropy_body(p_ref, mi_ref, hm_ref, hj_ref):
    eps = 1e-10
    mi_a = jnp.float32(0.0)
    hm_a = jnp.float32(0.0)
    hj_a = jnp.float32(0.0)
    for t in range(T):
        sls = _BY_T[t]
        j = p_ref[sls[0]]
        for sl in sls[1:]:
            j = j + p_ref[sl]
        total = jnp.sum(j)
        m0 = jnp.sum(j, axis=1)
        m1 = jnp.sum(j, axis=0)
        pj = j / total
        hj = -jnp.sum(pj * jnp.log2(pj + eps))
        p0 = m0 / total
        p1 = m1 / total
        hm = (-jnp.sum(p0 * jnp.log2(p0 + eps))
              - jnp.sum(p1 * jnp.log2(p1 + eps)))
        mi_a += (hm - hj) / hm
        hm_a += hm
        hj_a += hj
    mi_ref[0] = mi_a / T
    hm_ref[0] = hm_a / T
    hj_ref[0] = hj_a / T


def _entropy(parts):
    return pl.pallas_call(
        _tc_entropy_body,
        out_shape=[jax.ShapeDtypeStruct((1,), jnp.float32)] * 3,
        out_specs=[pl.BlockSpec(memory_space=pltpu.SMEM)] * 3,
    )(parts)


def kernel(inputs):
    xt = jnp.transpose(inputs, (0, 2, 1)).astype(jnp.uint32)
    parts = _sc_histogram(xt)
    mi, hm, hj = _entropy(parts.reshape(NSLOTS, NB, NB))
    return (mi[0], hm[0], hj[0])


# speedup vs baseline: 3435.9619x; 1.0563x over previous
import jax
import jax.numpy as jnp
from jax import lax
from jax.experimental import pallas as pl
from jax.experimental.pallas import tpu as pltpu
from jax.experimental.pallas import tpu_sc as plsc

T = 10
P = 1048576
S = 2
NB = 256
NBINS = NB * NB

NC = 2
NS = 16
CHUNK = 16384
H = P // NC
W = T * H
RANGE = W // NS
NSLOTS = NC * NS * 2

assert H % CHUNK == 0 and RANGE % CHUNK == 0


def _slot_schedule():
    by_t = {t: [] for t in range(T)}
    for c in range(NC):
        for s in range(NS):
            lo = s * RANGE
            t0 = lo // H
            r0 = lo - t0 * H
            len0 = min(RANGE, H - r0)
            wid = c * NS + s
            by_t[t0].append(wid * 2)
            if RANGE - len0 > 0:
                by_t[t0 + 1].append(wid * 2 + 1)
    return by_t


_BY_T = _slot_schedule()


def _i32(x):
    return jnp.asarray(x, jnp.int32)


def _sc_body(x_hbm, out_hbm, hist, abuf, bbuf):
    c = _i32(lax.axis_index("c"))
    s = _i32(lax.axis_index("s"))
    lo = s * _i32(RANGE)
    t0 = lo // _i32(H)
    r0 = lo - t0 * _i32(H)
    len0 = jnp.minimum(_i32(RANGE), _i32(H) - r0)
    n0 = len0 // _i32(CHUNK)
    n1 = (_i32(RANGE) - len0) // _i32(CHUNK)
    base_col = c * _i32(H)
    wid = c * _i32(NS) + s

    ones_f = jnp.ones((16,), jnp.float32)
    zeros_f = jnp.zeros((16,), jnp.float32)

    def zero_hist():
        def zbody(i, carry):
            off = i * _i32(64)
            for u in range(4):
                hist[pl.ds(off + _i32(u * 16), 16)] = zeros_f
            return carry

        lax.fori_loop(_i32(0), _i32(NBINS // 64), zbody, _i32(0))

    def process_chunk(t, start):
        pltpu.sync_copy(x_hbm.at[t, _i32(0), pl.ds(start, CHUNK)], abuf)
        pltpu.sync_copy(x_hbm.at[t, _i32(1), pl.ds(start, CHUNK)], bbuf)

        def ibody(i, carry):
            off = i * _i32(64)
            for u in range(4):
                o = off + _i32(u * 16)
                va = plsc.bitcast(abuf[pl.ds(o, 16)], jnp.int32)
                vb = plsc.bitcast(bbuf[pl.ds(o, 16)], jnp.int32)
                idx = va * _i32(NB) + vb
                plsc.addupdate_scatter(hist, [idx], ones_f)
            return carry

        lax.fori_loop(_i32(0), _i32(CHUNK // 64), ibody, _i32(0))

    def run_segment(t, seg_start, nchunks, slot):
        zero_hist()

        def chunk_body(k, carry):
            process_chunk(t, seg_start + k * _i32(CHUNK))
            return carry

        lax.fori_loop(_i32(0), nchunks, chunk_body, _i32(0))
        pltpu.sync_copy(hist, out_hbm.at[slot])

    run_segment(t0, base_col + r0, n0, wid * _i32(2))

    @pl.when(n1 > 0)
    def _():
        run_segment(t0 + _i32(1), base_col, n1, wid * _i32(2) + _i32(1))


def _sc_histogram(xw):
    mesh = plsc.VectorSubcoreMesh(
        core_axis_name="c", subcore_axis_name="s",
        num_cores=NC, num_subcores=NS)
    cp = pltpu.CompilerParams(
        needs_layout_passes=False, use_tc_tiling_on_sc=False)
    f = pl.kernel(
        _sc_body,
        out_type=jax.ShapeDtypeStruct((NSLOTS, NBINS), jnp.float32),
        mesh=mesh,
        scratch_types=[
            pltpu.VMEM((NBINS,), jnp.float32),
            pltpu.VMEM((CHUNK,), jnp.uint32),
            pltpu.VMEM((CHUNK,), jnp.uint32),
        ],
        compiler_params=cp,
    )
    return f(xw)


def _tc_entropy_body(p_ref, mi_ref, hm_ref, hj_ref):
    eps = 1e-10
    mi_a = jnp.float32(0.0)
    hm_a = jnp.float32(0.0)
    hj_a = jnp.float32(0.0)
    for t in range(T):
        sls = _BY_T[t]
        j = p_ref[sls[0]]
        for sl in sls[1:]:
            j = j + p_ref[sl]
        total = jnp.sum(j)
        m0 = jnp.sum(j, axis=1)
        m1 = jnp.sum(j, axis=0)
        pj = j / total
        hj = -jnp.sum(pj * jnp.log2(pj + eps))
        p0 = m0 / total
        p1 = m1 / total
        hm = (-jnp.sum(p0 * jnp.log2(p0 + eps))
              - jnp.sum(p1 * jnp.log2(p1 + eps)))
        mi_a += (hm - hj) / hm
        hm_a += hm
        hj_a += hj
    mi_ref[0] = mi_a / T
    hm_ref[0] = hm_a / T
    hj_ref[0] = hj_a / T


def _entropy(parts):
    return pl.pallas_call(
        _tc_entropy_body,
        out_shape=[jax.ShapeDtypeStruct((1,), jnp.float32)] * 3,
        out_specs=[pl.BlockSpec(memory_space=pltpu.SMEM)] * 3,
    )(parts)


def kernel(inputs):
    xt = jnp.transpose(inputs, (0, 2, 1)).astype(jnp.uint32)
    parts = _sc_histogram(xt)
    mi, hm, hj = _entropy(parts.reshape(NSLOTS, NB, NB))
    return (mi[0], hm[0], hj[0])
